# baseline (speedup 1.0000x reference)
import functools

import jax
import jax.numpy as jnp
from jax import lax
from jax.experimental import pallas as pl
from jax.experimental.pallas import tpu as pltpu
from jax.experimental.pallas import tpu_sc as plsc

N, C, H, W = 4, 96, 224, 224
HW = H * W
NPLANES = N * C
NWORKERS = 32
PPW = NPLANES // NWORKERS
K = 3136
NCHUNKS = HW // K
STEPS = K // 16


def _precompute_tc(gx, gy):
    ROWS = HW // 128

    def body(gx_ref, gy_ref, i_ref, wx_ref, wy_ref):
        gxv = gx_ref[...]
        gyv = gy_ref[...]
        ix = (gxv + 1.0) * (0.5 * (W - 1))
        iy = (gyv + 1.0) * (0.5 * (H - 1))
        ix = jnp.clip(ix, 0.0, float(W - 1))
        iy = jnp.clip(iy, 0.0, float(H - 1))
        ix0 = jnp.minimum(jnp.floor(ix), float(W - 2))
        iy0 = jnp.minimum(jnp.floor(iy), float(H - 2))
        wx_ref[...] = ix - ix0
        wy_ref[...] = iy - iy0
        i_ref[...] = iy0.astype(jnp.int32) * W + ix0.astype(jnp.int32)

    out_shape = (
        jax.ShapeDtypeStruct((N, ROWS, 128), jnp.int32),
        jax.ShapeDtypeStruct((N, ROWS, 128), jnp.float32),
        jax.ShapeDtypeStruct((N, ROWS, 128), jnp.float32),
    )
    i00, wx, wy = pl.pallas_call(body, out_shape=out_shape)(
        gx.reshape(N, ROWS, 128), gy.reshape(N, ROWS, 128)
    )
    return i00.reshape(N, HW), wx.reshape(N, HW), wy.reshape(N, HW)


def _sc_sample(inp_flat, i00, wx, wy):
    mesh = plsc.VectorSubcoreMesh(core_axis_name="c", subcore_axis_name="s")

    @functools.partial(
        pl.kernel,
        out_type=jax.ShapeDtypeStruct((NPLANES * HW,), jnp.float32),
        mesh=mesh,
        scratch_types=[
            pltpu.VMEM((HW,), jnp.float32),
            pltpu.VMEM((K,), jnp.int32),
            pltpu.VMEM((K,), jnp.float32),
            pltpu.VMEM((K,), jnp.float32),
            pltpu.VMEM((K,), jnp.float32),
        ],
        compiler_params=pltpu.CompilerParams(needs_layout_passes=False),
    )
    def body(inp_hbm, idx_hbm, wx_hbm, wy_hbm, out_hbm,
             plane_v, idx_v, wx_v, wy_v, out_v):
        wid = lax.axis_index("s") * 2 + lax.axis_index("c")
        n = wid // (NWORKERS // N)

        def plane_body(p, carry):
            plane_id = wid * PPW + p
            pltpu.sync_copy(inp_hbm.at[pl.ds(plane_id * HW, HW)], plane_v)

            def chunk_body(ch, carry2):
                off = ch * K
                goff = n * HW + off
                pltpu.sync_copy(idx_hbm.at[pl.ds(goff, K)], idx_v)
                pltpu.sync_copy(wx_hbm.at[pl.ds(goff, K)], wx_v)
                pltpu.sync_copy(wy_hbm.at[pl.ds(goff, K)], wy_v)

                def step(i, carry3):
                    s = i * 16
                    idx = idx_v[pl.ds(s, 16)]
                    fx1 = wx_v[pl.ds(s, 16)]
                    fy1 = wy_v[pl.ds(s, 16)]
                    v00 = plsc.load_gather(plane_v, [idx])
                    v01 = plsc.load_gather(plane_v, [idx + 1])
                    v10 = plsc.load_gather(plane_v, [idx + W])
                    v11 = plsc.load_gather(plane_v, [idx + (W + 1)])
                    fx0 = 1.0 - fx1
                    top = v00 * fx0 + v01 * fx1
                    bot = v10 * fx0 + v11 * fx1
                    out_v[pl.ds(s, 16)] = top * (1.0 - fy1) + bot * fy1
                    return carry3

                lax.fori_loop(0, STEPS, step, 0)
                pltpu.sync_copy(out_v, out_hbm.at[pl.ds(plane_id * HW + off, K)])
                return carry2

            lax.fori_loop(0, NCHUNKS, chunk_body, 0)
            return carry

        lax.fori_loop(0, PPW, plane_body, 0)

    return body(inp_flat, i00, wx, wy)


def kernel(input, grid):
    gx = grid[..., 0].reshape(N, HW)
    gy = grid[..., 1].reshape(N, HW)
    i00, wx, wy = _precompute_tc(gx, gy)
    out = _sc_sample(input.reshape(NPLANES * HW), i00.reshape(N * HW),
                     wx.reshape(N * HW), wy.reshape(N * HW))
    return out.reshape(N, C, H, W)


# speedup vs baseline: 2.6556x; 2.6556x over previous
import functools

import jax
import jax.numpy as jnp
from jax import lax
from jax.experimental import pallas as pl
from jax.experimental.pallas import tpu as pltpu
from jax.experimental.pallas import tpu_sc as plsc

N, C, H, W = 4, 96, 224, 224
HW = H * W
NPLANES = N * C
NWORKERS = 32
PPW = NPLANES // NWORKERS
NPAIRS = PPW // 2
K = 1792
NCHUNKS = HW // K
STEPS = K // 16
ITERS = NCHUNKS // 2


def _precompute_tc(gx, gy):
    ROWS = HW // 128

    def body(gx_ref, gy_ref, i_ref, wx_ref, wy_ref):
        gxv = gx_ref[...]
        gyv = gy_ref[...]
        ix = (gxv + 1.0) * (0.5 * (W - 1))
        iy = (gyv + 1.0) * (0.5 * (H - 1))
        ix = jnp.clip(ix, 0.0, float(W - 1))
        iy = jnp.clip(iy, 0.0, float(H - 1))
        ix0 = jnp.minimum(jnp.floor(ix), float(W - 2))
        iy0 = jnp.minimum(jnp.floor(iy), float(H - 2))
        wx_ref[...] = ix - ix0
        wy_ref[...] = iy - iy0
        i_ref[...] = iy0.astype(jnp.int32) * W + ix0.astype(jnp.int32)

    out_shape = (
        jax.ShapeDtypeStruct((N, ROWS, 128), jnp.int32),
        jax.ShapeDtypeStruct((N, ROWS, 128), jnp.float32),
        jax.ShapeDtypeStruct((N, ROWS, 128), jnp.float32),
    )
    i00, wx, wy = pl.pallas_call(body, out_shape=out_shape)(
        gx.reshape(N, ROWS, 128), gy.reshape(N, ROWS, 128)
    )
    return i00.reshape(N * HW), wx.reshape(N * HW), wy.reshape(N * HW)


def _sc_sample(inp_flat, i00, wx, wy):
    mesh = plsc.VectorSubcoreMesh(core_axis_name="c", subcore_axis_name="s")

    @functools.partial(
        pl.kernel,
        out_type=jax.ShapeDtypeStruct((NPLANES * HW,), jnp.float32),
        mesh=mesh,
        scratch_types=[
            pltpu.VMEM((HW,), jnp.float32),
            pltpu.VMEM((HW,), jnp.float32),
            pltpu.VMEM((2, K), jnp.int32),
            pltpu.VMEM((2, K), jnp.float32),
            pltpu.VMEM((2, K), jnp.float32),
            pltpu.VMEM((2, K), jnp.float32),
            pltpu.VMEM((2, K), jnp.float32),
            pltpu.SemaphoreType.DMA,
            pltpu.SemaphoreType.DMA,
            pltpu.SemaphoreType.DMA,
            pltpu.SemaphoreType.DMA,
            pltpu.SemaphoreType.DMA,
        ],
        compiler_params=pltpu.CompilerParams(needs_layout_passes=False),
    )
    def body(inp_hbm, idx_hbm, wx_hbm, wy_hbm, out_hbm,
             pl0_v, pl1_v, idx_v, wx_v, wy_v, out0_v, out1_v,
             sem_plane, sem_c0, sem_c1, sem_o0, sem_o1):
        wid = lax.axis_index("c") * 16 + lax.axis_index("s")
        n = wid // (NWORKERS // N)
        gbase = n * HW
        sem_c = (sem_c0, sem_c1)
        sem_o = (sem_o0, sem_o1)
        out_bufs = (out0_v, out1_v)

        def start_chunk(ch, s):
            off = gbase + ch * K
            pltpu.async_copy(idx_hbm.at[pl.ds(off, K)], idx_v.at[s], sem_c[s])
            pltpu.async_copy(wx_hbm.at[pl.ds(off, K)], wx_v.at[s], sem_c[s])
            pltpu.async_copy(wy_hbm.at[pl.ds(off, K)], wy_v.at[s], sem_c[s])

        def wait_chunk(ch, s):
            off = gbase + ch * K
            pltpu.make_async_copy(idx_hbm.at[pl.ds(off, K)], idx_v.at[s], sem_c[s]).wait()
            pltpu.make_async_copy(wx_hbm.at[pl.ds(off, K)], wx_v.at[s], sem_c[s]).wait()
            pltpu.make_async_copy(wy_hbm.at[pl.ds(off, K)], wy_v.at[s], sem_c[s]).wait()

        def start_out(p0_id, p1_id, ch, s):
            pltpu.async_copy(out0_v.at[s], out_hbm.at[pl.ds(p0_id * HW + ch * K, K)], sem_o[s])
            pltpu.async_copy(out1_v.at[s], out_hbm.at[pl.ds(p1_id * HW + ch * K, K)], sem_o[s])

        def wait_out(p0_id, p1_id, ch, s):
            pltpu.make_async_copy(out0_v.at[s], out_hbm.at[pl.ds(p0_id * HW + ch * K, K)], sem_o[s]).wait()
            pltpu.make_async_copy(out1_v.at[s], out_hbm.at[pl.ds(p1_id * HW + ch * K, K)], sem_o[s]).wait()

        def compute_chunk(s):
            @plsc.parallel_loop(0, STEPS, 1, unroll=4)
            def _(i):
                sl = i * 16
                idx = idx_v[s, pl.ds(sl, 16)]
                fx1 = wx_v[s, pl.ds(sl, 16)]
                fy1 = wy_v[s, pl.ds(sl, 16)]
                fx0 = 1.0 - fx1
                fy0 = 1.0 - fy1
                i01 = idx + 1
                i10 = idx + W
                i11 = idx + (W + 1)
                for pv, ov in ((pl0_v, out0_v), (pl1_v, out1_v)):
                    v00 = plsc.load_gather(pv, [idx])
                    v01 = plsc.load_gather(pv, [i01])
                    v10 = plsc.load_gather(pv, [i10])
                    v11 = plsc.load_gather(pv, [i11])
                    top = v00 * fx0 + v01 * fx1
                    bot = v10 * fx0 + v11 * fx1
                    ov[s, pl.ds(sl, 16)] = top * fy0 + bot * fy1

        def pair_body(p, carry):
            p0_id = wid * PPW + 2 * p
            p1_id = p0_id + 1
            pltpu.async_copy(inp_hbm.at[pl.ds(p0_id * HW, HW)], pl0_v, sem_plane)
            pltpu.async_copy(inp_hbm.at[pl.ds(p1_id * HW, HW)], pl1_v, sem_plane)
            start_chunk(0, 0)

            @pl.when(p > 0)
            def _():
                q0 = p0_id - 2
                q1 = q0 + 1
                wait_out(q0, q1, NCHUNKS - 2, 0)
                wait_out(q0, q1, NCHUNKS - 1, 1)

            pltpu.make_async_copy(inp_hbm.at[pl.ds(p0_id * HW, HW)], pl0_v, sem_plane).wait()
            pltpu.make_async_copy(inp_hbm.at[pl.ds(p1_id * HW, HW)], pl1_v, sem_plane).wait()

            def chunk_iter(it, carry2):
                ch0 = 2 * it
                ch1 = ch0 + 1
                start_chunk(ch1, 1)
                wait_chunk(ch0, 0)

                @pl.when(it > 0)
                def _():
                    wait_out(p0_id, p1_id, ch0 - 2, 0)

                compute_chunk(0)
                start_out(p0_id, p1_id, ch0, 0)
                @pl.when(ch1 + 1 < NCHUNKS)
                def _():
                    start_chunk(ch1 + 1, 0)

                wait_chunk(ch1, 1)

                @pl.when(it > 0)
                def _():
                    wait_out(p0_id, p1_id, ch1 - 2, 1)

                compute_chunk(1)
                start_out(p0_id, p1_id, ch1, 1)
                return carry2

            lax.fori_loop(0, ITERS, chunk_iter, 0)
            return carry

        lax.fori_loop(0, NPAIRS, pair_body, 0)
        q0 = wid * PPW + PPW - 2
        q1 = q0 + 1
        wait_out(q0, q1, NCHUNKS - 2, 0)
        wait_out(q0, q1, NCHUNKS - 1, 1)

    return body(inp_flat, i00, wx, wy)


def kernel(input, grid):
    gx = grid[..., 0].reshape(N, HW)
    gy = grid[..., 1].reshape(N, HW)
    i00, wx, wy = _precompute_tc(gx, gy)
    out = _sc_sample(input.reshape(NPLANES * HW), i00, wx, wy)
    return out.reshape(N, C, H, W)
